# baseline (speedup 1.0000x reference)
import functools

import jax
import jax.numpy as jnp
import numpy as np
from jax import lax
from jax.experimental import pallas as pl
from jax.experimental.pallas import tpu as pltpu
import jax.experimental.pallas.tpu_sc as plsc

N = 10000
E = 320000
D = 128

_NS = 16
_STRIPE = 632
_LAST = N - _STRIPE * (_NS - 1)
_CHUNK = 128
_CPT = 160
_NCHUNK = _CPT * _NS
_EPAD = _NCHUNK * _CHUNK
_NJUNK = 8
_CPD = _NCHUNK // 32
_G = 32

_sc_mesh = plsc.VectorSubcoreMesh(core_axis_name="c", subcore_axis_name="s")


@functools.partial(
    pl.kernel,
    out_type=[
        jax.ShapeDtypeStruct((N + _NJUNK,), jnp.float32),
        jax.ShapeDtypeStruct((N + _NJUNK,), jnp.float32),
    ],
    mesh=_sc_mesh,
    scratch_types=[
        pltpu.VMEM_SHARED((N + _NJUNK,), jnp.float32),
        pltpu.VMEM((_CPD, _CHUNK), jnp.int32),
        pltpu.VMEM((_CHUNK,), jnp.float32),
        pltpu.SemaphoreType.DMA,
    ],
)
def _deg_kernel(dst2_hbm, zdeg_hbm, ones_hbm, degp0_hbm, degp1_hbm,
                deg_sh, dstv_all, onesv, semd):
    cid = lax.axis_index("c")
    sid = lax.axis_index("s")

    @pl.when(sid == 0)
    def _():
        pltpu.sync_copy(zdeg_hbm, deg_sh)

    pltpu.sync_copy(ones_hbm, onesv)
    rowbase = cid * (_NCHUNK // 2) + sid * _CPD
    pltpu.sync_copy(dst2_hbm.at[pl.ds(rowbase, _CPD)], dstv_all)
    plsc.subcore_barrier()

    def fire(i, carry):
        pltpu.async_copy(onesv, deg_sh.at[dstv_all.at[i]], semd, add=True)
        return carry

    lax.fori_loop(0, _CPD, fire, 0)

    def drain(i, carry):
        pltpu.make_async_copy(onesv, deg_sh.at[dstv_all.at[i]], semd).wait()
        return carry

    lax.fori_loop(0, _CPD, drain, 0)
    plsc.subcore_barrier()

    @pl.when(sid == 0)
    def _():
        @pl.when(cid == 0)
        def _():
            pltpu.sync_copy(deg_sh, degp0_hbm)

        @pl.when(cid == 1)
        def _():
            pltpu.sync_copy(deg_sh, degp1_hbm)


@functools.partial(
    pl.kernel,
    out_type=[
        jax.ShapeDtypeStruct((N, D), jnp.float32),
        jax.ShapeDtypeStruct((N, D), jnp.float32),
    ],
    mesh=_sc_mesh,
    scratch_types=[
        pltpu.VMEM_SHARED((N + _NJUNK, D), jnp.float32),
        pltpu.VMEM((_G, _CHUNK), jnp.int32),
        pltpu.VMEM((_G, _CHUNK), jnp.int32),
        pltpu.VMEM((_CHUNK, D), jnp.float32),
        pltpu.VMEM((_CHUNK, D), jnp.float32),
        pltpu.SemaphoreType.DMA,
        pltpu.SemaphoreType.DMA,
        pltpu.SemaphoreType.DMA,
        pltpu.SemaphoreType.DMA,
    ],
)
def _scatter_kernel(tg_hbm, tu_hbm, src2_hbm, dst2_hbm, zrow_hbm,
                    accg_hbm, accu_hbm,
                    acc_sh, srcv_g, dstv_g, rows0, rows1,
                    semg0, semg1, sems0, sems1):
    cid = lax.axis_index("c")
    sid = lax.axis_index("s")

    def stripe_pieces(fn):
        base = sid * _STRIPE
        for j in range(4):
            fn(base + j * 128, 128)

        @pl.when(sid < _NS - 1)
        def _():
            fn(base + 512, 120)

        @pl.when(sid == _NS - 1)
        def _():
            fn(base + 512, 8)

    stripe_pieces(lambda row0, size: pltpu.sync_copy(
        zrow_hbm.at[pl.ds(0, size)], acc_sh.at[pl.ds(row0, size)]))

    plsc.subcore_barrier()

    def run(table_hbm):
        tilebase = sid * _CPT
        for g in range(_CPT // _G):
            gb = tilebase + g * _G
            pltpu.sync_copy(src2_hbm.at[pl.ds(gb, _G)], srcv_g)
            pltpu.sync_copy(dst2_hbm.at[pl.ds(gb, _G)], dstv_g)
            pltpu.async_copy(table_hbm.at[srcv_g.at[0]], rows0, semg0)

            def pair(p, carry):
                i0 = 2 * p
                i1 = i0 + 1
                pltpu.make_async_copy(table_hbm.at[srcv_g.at[i0]], rows0,
                                      semg0).wait()
                s0 = pltpu.async_copy(rows0, acc_sh.at[dstv_g.at[i0]], sems0,
                                      add=True)

                @pl.when(p > 0)
                def _():
                    pltpu.make_async_copy(rows1, acc_sh.at[dstv_g.at[i1 - 2]],
                                          sems1).wait()

                g1 = pltpu.async_copy(table_hbm.at[srcv_g.at[i1]], rows1,
                                      semg1)
                g1.wait()
                s1 = pltpu.async_copy(rows1, acc_sh.at[dstv_g.at[i1]], sems1,
                                      add=True)
                s0.wait()

                @pl.when(p < _G // 2 - 1)
                def _():
                    pltpu.async_copy(table_hbm.at[srcv_g.at[i0 + 2]], rows0,
                                     semg0)

                return carry

            lax.fori_loop(0, _G // 2, pair, 0)
            pltpu.make_async_copy(rows1, acc_sh.at[dstv_g.at[_G - 1]],
                                  sems1).wait()

    @pl.when(cid == 0)
    def _():
        run(tg_hbm)

    @pl.when(cid == 1)
    def _():
        run(tu_hbm)

    plsc.subcore_barrier()

    def writeout(out_hbm):
        stripe_pieces(lambda row0, size: pltpu.sync_copy(
            acc_sh.at[pl.ds(row0, size)], out_hbm.at[pl.ds(row0, size)]))

    @pl.when(cid == 0)
    def _():
        writeout(accg_hbm)

    @pl.when(cid == 1)
    def _():
        writeout(accu_hbm)


def _prep_a_body(x_ref, w_ref, b_ref, h_ref, tu_ref):
    h = jnp.dot(x_ref[...], w_ref[...], preferred_element_type=jnp.float32)
    h = h + b_ref[...]
    nrm2 = jnp.sum(h * h, axis=1, keepdims=True)
    inv_n = lax.rsqrt(jnp.maximum(nrm2, 1e-24))
    h_ref[...] = h
    tu_ref[...] = h * inv_n


_prep_a_call = pl.pallas_call(
    _prep_a_body,
    out_shape=[
        jax.ShapeDtypeStruct((N, D), jnp.float32),
        jax.ShapeDtypeStruct((N, D), jnp.float32),
    ],
)


def _prep_b_body(h_ref, d0_ref, d1_ref, tg_ref, r_ref):
    deg = d0_ref[pl.ds(0, N)] + d1_ref[pl.ds(0, N)]
    r = lax.rsqrt(jnp.maximum(deg, 1.0))
    tg_ref[...] = h_ref[...] * r
    r_ref[...] = r


_prep_b_call = pl.pallas_call(
    _prep_b_body,
    out_shape=[
        jax.ShapeDtypeStruct((N, D), jnp.float32),
        jax.ShapeDtypeStruct((N, 1), jnp.float32),
    ],
)


def _finish_body(accg_ref, accu_ref, tu_ref, r_ref, x_ref, g_ref, be_ref,
                 out_ref, gcd_ref):
    agg = r_ref[...] * accg_ref[...]
    mean = jnp.mean(agg, axis=0, keepdims=True)
    cent = agg - mean
    var = jnp.mean(cent * cent, axis=0, keepdims=True)
    y = cent * lax.rsqrt(var + 1e-5) * g_ref[...] + be_ref[...]
    out_ref[...] = jnp.maximum(y, 0.0) + x_ref[...]
    cos_sum = jnp.sum(tu_ref[...] * accu_ref[...])
    gcd_ref[...] = jnp.reshape(1.0 - cos_sum * (1.0 / E), (1, 1))


_finish_call = pl.pallas_call(
    _finish_body,
    out_shape=[
        jax.ShapeDtypeStruct((N, D), jnp.float32),
        jax.ShapeDtypeStruct((1, 1), jnp.float32),
    ],
)


_NPAD = _EPAD - E
_PAD_SRC = jnp.asarray((np.arange(_NPAD, dtype=np.int32) * 131) % N)
_PAD_DST = jnp.asarray(N + (np.arange(_NPAD, dtype=np.int32) % _NJUNK))
_ZDEG = jnp.zeros((N + _NJUNK,), jnp.float32)
_ONES = jnp.ones((_CHUNK,), jnp.float32)
_ZROW = jnp.zeros((_STRIPE, D), jnp.float32)


def kernel(x, edge_index, W, b, gamma, beta):
    src = edge_index[0]
    dst = edge_index[1]
    src2 = jnp.concatenate([src, _PAD_SRC]).reshape(_NCHUNK, _CHUNK)
    dst2 = jnp.concatenate([dst, _PAD_DST]).reshape(_NCHUNK, _CHUNK)

    degp0, degp1 = _deg_kernel(dst2, _ZDEG, _ONES)
    h, tu = _prep_a_call(x, W, b.reshape(1, D))
    tg, r = _prep_b_call(h, degp0.reshape(N + _NJUNK, 1),
                         degp1.reshape(N + _NJUNK, 1))
    accg, accu = _scatter_kernel(tg, tu, src2, dst2, _ZROW)
    out, gcd = _finish_call(accg, accu, tu, r, x,
                            gamma.reshape(1, D), beta.reshape(1, D))
    return out, gcd[0, 0]


# speedup vs baseline: 1.0208x; 1.0208x over previous
import functools

import jax
import jax.numpy as jnp
import numpy as np
from jax import lax
from jax.experimental import pallas as pl
from jax.experimental.pallas import tpu as pltpu
import jax.experimental.pallas.tpu_sc as plsc

N = 10000
E = 320000
D = 128

_NS = 16
_STRIPE = 632
_LAST = N - _STRIPE * (_NS - 1)
_CHUNK = 128
_CPT = 160
_NCHUNK = _CPT * _NS
_EPAD = _NCHUNK * _CHUNK
_NJUNK = 8
_CPD = _NCHUNK // 32
_G = 16
_NGR = _CPT // _G

_sc_mesh = plsc.VectorSubcoreMesh(core_axis_name="c", subcore_axis_name="s")


@functools.partial(
    pl.kernel,
    out_type=[
        jax.ShapeDtypeStruct((N + _NJUNK,), jnp.float32),
        jax.ShapeDtypeStruct((N + _NJUNK,), jnp.float32),
    ],
    mesh=_sc_mesh,
    scratch_types=[
        pltpu.VMEM_SHARED((N + _NJUNK,), jnp.float32),
        pltpu.VMEM((_CPD, _CHUNK), jnp.int32),
        pltpu.VMEM((_CHUNK,), jnp.float32),
        pltpu.SemaphoreType.DMA,
    ],
)
def _deg_kernel(dst2_hbm, zdeg_hbm, ones_hbm, degp0_hbm, degp1_hbm,
                deg_sh, dstv_all, onesv, semd):
    cid = lax.axis_index("c")
    sid = lax.axis_index("s")

    @pl.when(sid == 0)
    def _():
        pltpu.sync_copy(zdeg_hbm, deg_sh)

    pltpu.sync_copy(ones_hbm, onesv)
    rowbase = cid * (_NCHUNK // 2) + sid * _CPD
    pltpu.sync_copy(dst2_hbm.at[pl.ds(rowbase, _CPD)], dstv_all)
    plsc.subcore_barrier()

    def fire(i, carry):
        pltpu.async_copy(onesv, deg_sh.at[dstv_all.at[i]], semd, add=True)
        return carry

    lax.fori_loop(0, _CPD, fire, 0)

    def drain(i, carry):
        pltpu.make_async_copy(onesv, deg_sh.at[dstv_all.at[i]], semd).wait()
        return carry

    lax.fori_loop(0, _CPD, drain, 0)
    plsc.subcore_barrier()

    @pl.when(sid == 0)
    def _():
        @pl.when(cid == 0)
        def _():
            pltpu.sync_copy(deg_sh, degp0_hbm)

        @pl.when(cid == 1)
        def _():
            pltpu.sync_copy(deg_sh, degp1_hbm)


@functools.partial(
    pl.kernel,
    out_type=[
        jax.ShapeDtypeStruct((N, D), jnp.float32),
        jax.ShapeDtypeStruct((N, D), jnp.float32),
    ],
    mesh=_sc_mesh,
    scratch_types=[
        pltpu.VMEM_SHARED((N + _NJUNK, D), jnp.float32),
        pltpu.VMEM((3, _G, _CHUNK), jnp.int32),
        pltpu.VMEM((3, _G, _CHUNK), jnp.int32),
        pltpu.VMEM((_CHUNK, D), jnp.float32),
        pltpu.VMEM((_CHUNK, D), jnp.float32),
        pltpu.SemaphoreType.DMA,
        pltpu.SemaphoreType.DMA,
        pltpu.SemaphoreType.DMA,
        pltpu.SemaphoreType.DMA,
        pltpu.SemaphoreType.DMA,
    ],
)
def _scatter_kernel(tg_hbm, tu_hbm, src2_hbm, dst2_hbm, zrow_hbm,
                    accg_hbm, accu_hbm,
                    acc_sh, srcv3, dstv3, rows0, rows1,
                    semg0, semg1, sems0, sems1, semi):
    cid = lax.axis_index("c")
    sid = lax.axis_index("s")

    def stripe_pieces(fn):
        base = sid * _STRIPE
        for j in range(4):
            fn(base + j * 128, 128)

        @pl.when(sid < _NS - 1)
        def _():
            fn(base + 512, 120)

        @pl.when(sid == _NS - 1)
        def _():
            fn(base + 512, 8)

    stripe_pieces(lambda row0, size: pltpu.sync_copy(
        zrow_hbm.at[pl.ds(0, size)], acc_sh.at[pl.ds(row0, size)]))

    plsc.subcore_barrier()

    def run(table_hbm):
        tilebase = sid * _CPT

        def idx_load(grp, slot, copy):
            gb = tilebase + grp * _G
            copy(src2_hbm.at[pl.ds(gb, _G)], srcv3.at[slot])
            copy(dst2_hbm.at[pl.ds(gb, _G)], dstv3.at[slot])

        idx_load(0, 0, pltpu.sync_copy)
        idx_load(1, 1, lambda s, d: pltpu.async_copy(s, d, semi))
        pltpu.async_copy(table_hbm.at[srcv3.at[0, 0]], rows0, semg0)

        def pair(p, carry):
            i0 = 2 * p
            i1 = i0 + 1
            sl = lax.rem(i0 // _G, 3)
            j0 = lax.rem(i0, _G)
            j1 = j0 + 1
            pltpu.make_async_copy(table_hbm.at[srcv3.at[sl, j0]], rows0,
                                  semg0).wait()
            s0 = pltpu.async_copy(rows0, acc_sh.at[dstv3.at[sl, j0]], sems0,
                                  add=True)

            @pl.when(p > 0)
            def _():
                ip = i1 - 2
                pltpu.make_async_copy(
                    rows1, acc_sh.at[dstv3.at[lax.rem(ip // _G, 3),
                                              lax.rem(ip, _G)]],
                    sems1).wait()

            g1 = pltpu.async_copy(table_hbm.at[srcv3.at[sl, j1]], rows1,
                                  semg1)
            g1.wait()
            s1 = pltpu.async_copy(rows1, acc_sh.at[dstv3.at[sl, j1]], sems1,
                                  add=True)
            s0.wait()

            @pl.when(p < _CPT // 2 - 1)
            def _():
                inext = i0 + 2
                gn = inext // _G
                sn = lax.rem(gn, 3)
                jn = lax.rem(inext, _G)

                @pl.when(jn == 0)
                def _():
                    idx_load(gn, sn,
                             lambda s, d: pltpu.make_async_copy(s, d,
                                                                semi).wait())

                    @pl.when(gn + 1 < _NGR)
                    def _():
                        idx_load(gn + 1, lax.rem(gn + 1, 3),
                                 lambda s, d: pltpu.async_copy(s, d, semi))

                pltpu.async_copy(table_hbm.at[srcv3.at[sn, jn]], rows0, semg0)

            return carry

        lax.fori_loop(0, _CPT // 2, pair, 0)
        pltpu.make_async_copy(
            rows1, acc_sh.at[dstv3.at[lax.rem(_NGR - 1, 3), _G - 1]],
            sems1).wait()

    @pl.when(cid == 0)
    def _():
        run(tg_hbm)

    @pl.when(cid == 1)
    def _():
        run(tu_hbm)

    plsc.subcore_barrier()

    def writeout(out_hbm):
        stripe_pieces(lambda row0, size: pltpu.sync_copy(
            acc_sh.at[pl.ds(row0, size)], out_hbm.at[pl.ds(row0, size)]))

    @pl.when(cid == 0)
    def _():
        writeout(accg_hbm)

    @pl.when(cid == 1)
    def _():
        writeout(accu_hbm)


def _prep_a_body(x_ref, w_ref, b_ref, h_ref, tu_ref):
    h = jnp.dot(x_ref[...], w_ref[...], preferred_element_type=jnp.float32)
    h = h + b_ref[...]
    nrm2 = jnp.sum(h * h, axis=1, keepdims=True)
    inv_n = lax.rsqrt(jnp.maximum(nrm2, 1e-24))
    h_ref[...] = h
    tu_ref[...] = h * inv_n


_prep_a_call = pl.pallas_call(
    _prep_a_body,
    out_shape=[
        jax.ShapeDtypeStruct((N, D), jnp.float32),
        jax.ShapeDtypeStruct((N, D), jnp.float32),
    ],
)


def _prep_b_body(h_ref, d0_ref, d1_ref, tg_ref, r_ref):
    deg = d0_ref[pl.ds(0, N)] + d1_ref[pl.ds(0, N)]
    r = lax.rsqrt(jnp.maximum(deg, 1.0))
    tg_ref[...] = h_ref[...] * r
    r_ref[...] = r


_prep_b_call = pl.pallas_call(
    _prep_b_body,
    out_shape=[
        jax.ShapeDtypeStruct((N, D), jnp.float32),
        jax.ShapeDtypeStruct((N, 1), jnp.float32),
    ],
)


def _finish_body(accg_ref, accu_ref, tu_ref, r_ref, x_ref, g_ref, be_ref,
                 out_ref, gcd_ref):
    agg = r_ref[...] * accg_ref[...]
    mean = jnp.mean(agg, axis=0, keepdims=True)
    cent = agg - mean
    var = jnp.mean(cent * cent, axis=0, keepdims=True)
    y = cent * lax.rsqrt(var + 1e-5) * g_ref[...] + be_ref[...]
    out_ref[...] = jnp.maximum(y, 0.0) + x_ref[...]
    cos_sum = jnp.sum(tu_ref[...] * accu_ref[...])
    gcd_ref[...] = jnp.reshape(1.0 - cos_sum * (1.0 / E), (1, 1))


_finish_call = pl.pallas_call(
    _finish_body,
    out_shape=[
        jax.ShapeDtypeStruct((N, D), jnp.float32),
        jax.ShapeDtypeStruct((1, 1), jnp.float32),
    ],
)


_NPAD = _EPAD - E
_PAD_SRC = jnp.asarray((np.arange(_NPAD, dtype=np.int32) * 131) % N)
_PAD_DST = jnp.asarray(N + (np.arange(_NPAD, dtype=np.int32) % _NJUNK))
_ZDEG = jnp.zeros((N + _NJUNK,), jnp.float32)
_ONES = jnp.ones((_CHUNK,), jnp.float32)
_ZROW = jnp.zeros((_STRIPE, D), jnp.float32)


def kernel(x, edge_index, W, b, gamma, beta):
    src = edge_index[0]
    dst = edge_index[1]
    src2 = jnp.concatenate([src, _PAD_SRC]).reshape(_NCHUNK, _CHUNK)
    dst2 = jnp.concatenate([dst, _PAD_DST]).reshape(_NCHUNK, _CHUNK)

    degp0, degp1 = _deg_kernel(dst2, _ZDEG, _ONES)
    h, tu = _prep_a_call(x, W, b.reshape(1, D))
    tg, r = _prep_b_call(h, degp0.reshape(N + _NJUNK, 1),
                         degp1.reshape(N + _NJUNK, 1))
    accg, accu = _scatter_kernel(tg, tu, src2, dst2, _ZROW)
    out, gcd = _finish_call(accg, accu, tu, r, x,
                            gamma.reshape(1, D), beta.reshape(1, D))
    return out, gcd[0, 0]


# speedup vs baseline: 1.0381x; 1.0170x over previous
import functools

import jax
import jax.numpy as jnp
import numpy as np
from jax import lax
from jax.experimental import pallas as pl
from jax.experimental.pallas import tpu as pltpu
import jax.experimental.pallas.tpu_sc as plsc

N = 10000
E = 320000
D = 128

_NS = 16
_STRIPE = 632
_LAST = N - _STRIPE * (_NS - 1)
_CHUNK = 128
_CPT = 160
_NCHUNK = _CPT * _NS
_EPAD = _NCHUNK * _CHUNK
_NJUNK = 8
_CPD = _NCHUNK // 32
_G = 16
_NGR = _CPT // _G

_sc_mesh = plsc.VectorSubcoreMesh(core_axis_name="c", subcore_axis_name="s")


@functools.partial(
    pl.kernel,
    out_type=[
        jax.ShapeDtypeStruct((N + _NJUNK,), jnp.float32),
        jax.ShapeDtypeStruct((N + _NJUNK,), jnp.float32),
    ],
    mesh=_sc_mesh,
    scratch_types=[
        pltpu.VMEM_SHARED((N + _NJUNK,), jnp.float32),
        pltpu.VMEM((_CPD, _CHUNK), jnp.int32),
        pltpu.VMEM((_CHUNK,), jnp.float32),
        pltpu.SemaphoreType.DMA,
    ],
)
def _deg_kernel(dst2_hbm, zdeg_hbm, ones_hbm, degp0_hbm, degp1_hbm,
                deg_sh, dstv_all, onesv, semd):
    cid = lax.axis_index("c")
    sid = lax.axis_index("s")

    @pl.when(sid == 0)
    def _():
        pltpu.sync_copy(zdeg_hbm, deg_sh)

    pltpu.sync_copy(ones_hbm, onesv)
    rowbase = cid * (_NCHUNK // 2) + sid * _CPD
    pltpu.sync_copy(dst2_hbm.at[pl.ds(rowbase, _CPD)], dstv_all)
    plsc.subcore_barrier()

    def fire(i, carry):
        pltpu.async_copy(onesv, deg_sh.at[dstv_all.at[i]], semd, add=True)
        return carry

    lax.fori_loop(0, _CPD, fire, 0)

    def drain(i, carry):
        pltpu.make_async_copy(onesv, deg_sh.at[dstv_all.at[i]], semd).wait()
        return carry

    lax.fori_loop(0, _CPD, drain, 0)
    plsc.subcore_barrier()

    @pl.when(sid == 0)
    def _():
        @pl.when(cid == 0)
        def _():
            pltpu.sync_copy(deg_sh, degp0_hbm)

        @pl.when(cid == 1)
        def _():
            pltpu.sync_copy(deg_sh, degp1_hbm)


@functools.partial(
    pl.kernel,
    out_type=[
        jax.ShapeDtypeStruct((N, D), jnp.float32),
        jax.ShapeDtypeStruct((N, D), jnp.float32),
    ],
    mesh=_sc_mesh,
    scratch_types=[
        pltpu.VMEM_SHARED((N + _NJUNK, D), jnp.float32),
        pltpu.VMEM((2, _G, _CHUNK), jnp.int32),
        pltpu.VMEM((3, 2 * _G, _CHUNK // 2), jnp.int32),
        pltpu.VMEM((_CHUNK, D), jnp.float32),
        pltpu.VMEM((_CHUNK, D), jnp.float32),
        pltpu.SemaphoreType.DMA,
        pltpu.SemaphoreType.DMA,
        pltpu.SemaphoreType.DMA,
        pltpu.SemaphoreType.DMA,
        pltpu.SemaphoreType.DMA,
        pltpu.SemaphoreType.DMA,
        pltpu.SemaphoreType.DMA,
        pltpu.SemaphoreType.DMA,
        pltpu.SemaphoreType.DMA,
    ],
)
def _scatter_kernel(tg_hbm, tu_hbm, src2_hbm, dst2h_hbm, zrow_hbm,
                    accg_hbm, accu_hbm,
                    acc_sh, srcv3, dstv3, rows0, rows1,
                    semg0a, semg0b, semg1a, semg1b,
                    sems0a, sems0b, sems1a, sems1b, semi):
    cid = lax.axis_index("c")
    sid = lax.axis_index("s")

    def stripe_pieces(fn):
        base = sid * _STRIPE
        for j in range(4):
            fn(base + j * 128, 128)

        @pl.when(sid < _NS - 1)
        def _():
            fn(base + 512, 120)

        @pl.when(sid == _NS - 1)
        def _():
            fn(base + 512, 8)

    stripe_pieces(lambda row0, size: pltpu.sync_copy(
        zrow_hbm.at[pl.ds(0, size)], acc_sh.at[pl.ds(row0, size)]))

    plsc.subcore_barrier()

    def run(table_hbm):
        tilebase = sid * _CPT
        H = _CHUNK // 2

        def idx_load(grp, copy):
            gb = tilebase + grp * _G
            copy(src2_hbm.at[pl.ds(gb, _G)], srcv3.at[grp % 2])
            copy(dst2h_hbm.at[pl.ds(2 * gb, 2 * _G)], dstv3.at[grp % 3])

        def gather_halves(sl, j, rowbuf, sa, sb):
            return (
                pltpu.make_async_copy(
                    table_hbm.at[srcv3.at[sl, j, pl.ds(0, H)]],
                    rowbuf.at[pl.ds(0, H)], sa),
                pltpu.make_async_copy(
                    table_hbm.at[srcv3.at[sl, j, pl.ds(H, H)]],
                    rowbuf.at[pl.ds(H, H)], sb),
            )

        def scatter_halves(sl, j, rowbuf, sa, sb):
            return (
                pltpu.make_async_copy(
                    rowbuf.at[pl.ds(0, H)], acc_sh.at[dstv3.at[sl, 2 * j]],
                    sa),
                pltpu.make_async_copy(
                    rowbuf.at[pl.ds(H, H)], acc_sh.at[dstv3.at[sl, 2 * j + 1]],
                    sb),
            )

        def fire_gather(sl, j, rowbuf, sa, sb):
            pltpu.async_copy(table_hbm.at[srcv3.at[sl, j, pl.ds(0, H)]],
                             rowbuf.at[pl.ds(0, H)], sa)
            pltpu.async_copy(table_hbm.at[srcv3.at[sl, j, pl.ds(H, H)]],
                             rowbuf.at[pl.ds(H, H)], sb)

        def fire_scatter(sl, j, rowbuf, sa, sb):
            pltpu.async_copy(rowbuf.at[pl.ds(0, H)],
                             acc_sh.at[dstv3.at[sl, 2 * j]], sa, add=True)
            pltpu.async_copy(rowbuf.at[pl.ds(H, H)],
                             acc_sh.at[dstv3.at[sl, 2 * j + 1]], sb, add=True)

        def wait2(descs):
            descs[0].wait()
            descs[1].wait()

        idx_load(0, pltpu.sync_copy)
        idx_load(1, lambda s, d: pltpu.async_copy(s, d, semi))
        fire_gather(0, 0, rows0, semg0a, semg0b)

        def pair(p, carry):
            i0 = 2 * p
            i1 = i0 + 1
            g2 = (i0 // _G) % 2
            g3 = (i0 // _G) % 3
            j0 = lax.rem(i0, _G)
            j1 = j0 + 1
            wait2(gather_halves(g2, j0, rows0, semg0a, semg0b))
            fire_scatter(g3, j0, rows0, sems0a, sems0b)

            @pl.when(p > 0)
            def _():
                ip = i1 - 2
                wait2(scatter_halves((ip // _G) % 3, lax.rem(ip, _G),
                                     rows1, sems1a, sems1b))

            fire_gather(g2, j1, rows1, semg1a, semg1b)
            wait2(gather_halves(g2, j1, rows1, semg1a, semg1b))
            fire_scatter(g3, j1, rows1, sems1a, sems1b)
            wait2(scatter_halves(g3, j0, rows0, sems0a, sems0b))

            @pl.when(p < _CPT // 2 - 1)
            def _():
                inext = i0 + 2
                gn = inext // _G
                jn = lax.rem(inext, _G)

                @pl.when(jn == 0)
                def _():
                    idx_load(gn,
                             lambda s, d: pltpu.make_async_copy(s, d,
                                                                semi).wait())

                    @pl.when(gn + 1 < _NGR)
                    def _():
                        idx_load(gn + 1,
                                 lambda s, d: pltpu.async_copy(s, d, semi))

                fire_gather(gn % 2, jn, rows0, semg0a, semg0b)

            return carry

        lax.fori_loop(0, _CPT // 2, pair, 0)
        wait2(scatter_halves((_NGR - 1) % 3, _G - 1, rows1,
                             sems1a, sems1b))

    @pl.when(cid == 0)
    def _():
        run(tg_hbm)

    @pl.when(cid == 1)
    def _():
        run(tu_hbm)

    plsc.subcore_barrier()

    def writeout(out_hbm):
        stripe_pieces(lambda row0, size: pltpu.sync_copy(
            acc_sh.at[pl.ds(row0, size)], out_hbm.at[pl.ds(row0, size)]))

    @pl.when(cid == 0)
    def _():
        writeout(accg_hbm)

    @pl.when(cid == 1)
    def _():
        writeout(accu_hbm)


def _prep_a_body(x_ref, w_ref, b_ref, h_ref, tu_ref):
    h = jnp.dot(x_ref[...], w_ref[...], preferred_element_type=jnp.float32)
    h = h + b_ref[...]
    nrm2 = jnp.sum(h * h, axis=1, keepdims=True)
    inv_n = lax.rsqrt(jnp.maximum(nrm2, 1e-24))
    h_ref[...] = h
    tu_ref[...] = h * inv_n


_prep_a_call = pl.pallas_call(
    _prep_a_body,
    out_shape=[
        jax.ShapeDtypeStruct((N, D), jnp.float32),
        jax.ShapeDtypeStruct((N, D), jnp.float32),
    ],
)


def _prep_b_body(h_ref, d0_ref, d1_ref, tg_ref, r_ref):
    deg = d0_ref[pl.ds(0, N)] + d1_ref[pl.ds(0, N)]
    r = lax.rsqrt(jnp.maximum(deg, 1.0))
    tg_ref[...] = h_ref[...] * r
    r_ref[...] = r


_prep_b_call = pl.pallas_call(
    _prep_b_body,
    out_shape=[
        jax.ShapeDtypeStruct((N, D), jnp.float32),
        jax.ShapeDtypeStruct((N, 1), jnp.float32),
    ],
)


def _finish_body(accg_ref, accu_ref, tu_ref, r_ref, x_ref, g_ref, be_ref,
                 out_ref, gcd_ref):
    agg = r_ref[...] * accg_ref[...]
    mean = jnp.mean(agg, axis=0, keepdims=True)
    cent = agg - mean
    var = jnp.mean(cent * cent, axis=0, keepdims=True)
    y = cent * lax.rsqrt(var + 1e-5) * g_ref[...] + be_ref[...]
    out_ref[...] = jnp.maximum(y, 0.0) + x_ref[...]
    cos_sum = jnp.sum(tu_ref[...] * accu_ref[...])
    gcd_ref[...] = jnp.reshape(1.0 - cos_sum * (1.0 / E), (1, 1))


_finish_call = pl.pallas_call(
    _finish_body,
    out_shape=[
        jax.ShapeDtypeStruct((N, D), jnp.float32),
        jax.ShapeDtypeStruct((1, 1), jnp.float32),
    ],
)


_NPAD = _EPAD - E
_PAD_SRC = jnp.asarray((np.arange(_NPAD, dtype=np.int32) * 131) % N)
_PAD_DST = jnp.asarray(N + (np.arange(_NPAD, dtype=np.int32) % _NJUNK))
_ZDEG = jnp.zeros((N + _NJUNK,), jnp.float32)
_ONES = jnp.ones((_CHUNK,), jnp.float32)
_ZROW = jnp.zeros((_STRIPE, D), jnp.float32)


def kernel(x, edge_index, W, b, gamma, beta):
    src = edge_index[0]
    dst = edge_index[1]
    src2 = jnp.concatenate([src, _PAD_SRC]).reshape(_NCHUNK, _CHUNK)
    dst2 = jnp.concatenate([dst, _PAD_DST]).reshape(_NCHUNK, _CHUNK)

    degp0, degp1 = _deg_kernel(dst2, _ZDEG, _ONES)
    h, tu = _prep_a_call(x, W, b.reshape(1, D))
    tg, r = _prep_b_call(h, degp0.reshape(N + _NJUNK, 1),
                         degp1.reshape(N + _NJUNK, 1))
    dst2h = dst2.reshape(_NCHUNK * 2, _CHUNK // 2)
    accg, accu = _scatter_kernel(tg, tu, src2, dst2h, _ZROW)
    out, gcd = _finish_call(accg, accu, tu, r, x,
                            gamma.reshape(1, D), beta.reshape(1, D))
    return out, gcd[0, 0]


# speedup vs baseline: 1.1036x; 1.0631x over previous
import functools

import jax
import jax.numpy as jnp
import numpy as np
from jax import lax
from jax.experimental import pallas as pl
from jax.experimental.pallas import tpu as pltpu
import jax.experimental.pallas.tpu_sc as plsc

N = 10000
E = 320000
D = 128

_NS = 16
_STRIPE = 632
_LAST = N - _STRIPE * (_NS - 1)
_CHUNK = 128
_CPT = 160
_NCHUNK = _CPT * _NS
_EPAD = _NCHUNK * _CHUNK
_NJUNK = 8
_CPD = _NCHUNK // 32
_G = 16
_NGR = _CPT // _G

_sc_mesh = plsc.VectorSubcoreMesh(core_axis_name="c", subcore_axis_name="s")


@functools.partial(
    pl.kernel,
    out_type=[
        jax.ShapeDtypeStruct((N + _NJUNK,), jnp.float32),
        jax.ShapeDtypeStruct((N + _NJUNK,), jnp.float32),
    ],
    mesh=_sc_mesh,
    scratch_types=[
        pltpu.VMEM_SHARED((N + _NJUNK,), jnp.float32),
        pltpu.VMEM((_CPD, _CHUNK), jnp.int32),
        pltpu.VMEM((_CHUNK,), jnp.float32),
        pltpu.SemaphoreType.DMA,
    ],
)
def _deg_kernel(dst2_hbm, zdeg_hbm, ones_hbm, degp0_hbm, degp1_hbm,
                deg_sh, dstv_all, onesv, semd):
    cid = lax.axis_index("c")
    sid = lax.axis_index("s")

    @pl.when(sid == 0)
    def _():
        pltpu.sync_copy(zdeg_hbm, deg_sh)

    pltpu.sync_copy(ones_hbm, onesv)
    rowbase = cid * (_NCHUNK // 2) + sid * _CPD
    pltpu.sync_copy(dst2_hbm.at[pl.ds(rowbase, _CPD)], dstv_all)
    plsc.subcore_barrier()

    def fire(i, carry):
        pltpu.async_copy(onesv, deg_sh.at[dstv_all.at[i]], semd, add=True)
        return carry

    lax.fori_loop(0, _CPD, fire, 0)

    def drain(i, carry):
        pltpu.make_async_copy(onesv, deg_sh.at[dstv_all.at[i]], semd).wait()
        return carry

    lax.fori_loop(0, _CPD, drain, 0)
    plsc.subcore_barrier()

    @pl.when(sid == 0)
    def _():
        @pl.when(cid == 0)
        def _():
            pltpu.sync_copy(deg_sh, degp0_hbm)

        @pl.when(cid == 1)
        def _():
            pltpu.sync_copy(deg_sh, degp1_hbm)


@functools.partial(
    pl.kernel,
    out_type=[
        jax.ShapeDtypeStruct((N, D), jnp.float32),
        jax.ShapeDtypeStruct((N, D), jnp.float32),
    ],
    mesh=_sc_mesh,
    scratch_types=[
        pltpu.VMEM_SHARED((N + _NJUNK, D), jnp.float32),
        pltpu.VMEM((2, _G, _CHUNK), jnp.int32),
        pltpu.VMEM((3, 2 * _G, _CHUNK // 2), jnp.int32),
        pltpu.VMEM((_CHUNK, D), jnp.float32),
        pltpu.VMEM((_CHUNK, D), jnp.float32),
        pltpu.SemaphoreType.DMA,
        pltpu.SemaphoreType.DMA,
        pltpu.SemaphoreType.DMA,
        pltpu.SemaphoreType.DMA,
        pltpu.SemaphoreType.DMA,
        pltpu.SemaphoreType.DMA,
        pltpu.SemaphoreType.DMA,
        pltpu.SemaphoreType.DMA,
        pltpu.SemaphoreType.DMA,
    ],
)
def _scatter_kernel(tg_hbm, tu_hbm, src2_hbm, dst2h_hbm, zrow_hbm,
                    accg_hbm, accu_hbm,
                    acc_sh, srcv3, dstv3, rows0, rows1,
                    semg0a, semg0b, semg1a, semg1b,
                    sems0a, sems0b, sems1a, sems1b, semi):
    cid = lax.axis_index("c")
    sid = lax.axis_index("s")

    def stripe_pieces(fn):
        base = sid * _STRIPE
        for j in range(4):
            fn(base + j * 128, 128)

        @pl.when(sid < _NS - 1)
        def _():
            fn(base + 512, 120)

        @pl.when(sid == _NS - 1)
        def _():
            fn(base + 512, 8)

    stripe_pieces(lambda row0, size: pltpu.sync_copy(
        zrow_hbm.at[pl.ds(0, size)], acc_sh.at[pl.ds(row0, size)]))

    plsc.subcore_barrier()

    def run(table_hbm):
        tilebase = sid * _CPT
        H = _CHUNK // 2

        def idx_load(grp, copy):
            gb = tilebase + grp * _G
            copy(src2_hbm.at[pl.ds(gb, _G)], srcv3.at[grp % 2])
            copy(dst2h_hbm.at[pl.ds(2 * gb, 2 * _G)], dstv3.at[grp % 3])

        def gather_halves(sl, j, rowbuf, sa, sb):
            return (
                pltpu.make_async_copy(
                    table_hbm.at[srcv3.at[sl, j, pl.ds(0, H)]],
                    rowbuf.at[pl.ds(0, H)], sa),
                pltpu.make_async_copy(
                    table_hbm.at[srcv3.at[sl, j, pl.ds(H, H)]],
                    rowbuf.at[pl.ds(H, H)], sb),
            )

        def scatter_halves(sl, j, rowbuf, sa, sb):
            return (
                pltpu.make_async_copy(
                    rowbuf.at[pl.ds(0, H)], acc_sh.at[dstv3.at[sl, 2 * j]],
                    sa),
                pltpu.make_async_copy(
                    rowbuf.at[pl.ds(H, H)], acc_sh.at[dstv3.at[sl, 2 * j + 1]],
                    sb),
            )

        def fire_gather(sl, j, rowbuf, sa, sb):
            pltpu.async_copy(table_hbm.at[srcv3.at[sl, j, pl.ds(0, H)]],
                             rowbuf.at[pl.ds(0, H)], sa)
            pltpu.async_copy(table_hbm.at[srcv3.at[sl, j, pl.ds(H, H)]],
                             rowbuf.at[pl.ds(H, H)], sb)

        def fire_scatter(sl, j, rowbuf, sa, sb):
            pltpu.async_copy(rowbuf.at[pl.ds(0, H)],
                             acc_sh.at[dstv3.at[sl, 2 * j]], sa, add=True)
            pltpu.async_copy(rowbuf.at[pl.ds(H, H)],
                             acc_sh.at[dstv3.at[sl, 2 * j + 1]], sb, add=True)

        def wait2(descs):
            descs[0].wait()
            descs[1].wait()

        idx_load(0, pltpu.sync_copy)
        idx_load(1, lambda s, d: pltpu.async_copy(s, d, semi))
        fire_gather(0, 0, rows0, semg0a, semg0b)

        def pair(p, carry):
            i0 = 2 * p
            i1 = i0 + 1
            g2 = (i0 // _G) % 2
            g3 = (i0 // _G) % 3
            j0 = lax.rem(i0, _G)
            j1 = j0 + 1
            @pl.when(p > 0)
            def _():
                ip = i1 - 2
                wait2(scatter_halves((ip // _G) % 3, lax.rem(ip, _G),
                                     rows1, sems1a, sems1b))

            fire_gather(g2, j1, rows1, semg1a, semg1b)
            wait2(gather_halves(g2, j0, rows0, semg0a, semg0b))
            fire_scatter(g3, j0, rows0, sems0a, sems0b)
            wait2(gather_halves(g2, j1, rows1, semg1a, semg1b))
            fire_scatter(g3, j1, rows1, sems1a, sems1b)
            wait2(scatter_halves(g3, j0, rows0, sems0a, sems0b))

            @pl.when(p < _CPT // 2 - 1)
            def _():
                inext = i0 + 2
                gn = inext // _G
                jn = lax.rem(inext, _G)

                @pl.when(jn == 0)
                def _():
                    idx_load(gn,
                             lambda s, d: pltpu.make_async_copy(s, d,
                                                                semi).wait())

                    @pl.when(gn + 1 < _NGR)
                    def _():
                        idx_load(gn + 1,
                                 lambda s, d: pltpu.async_copy(s, d, semi))

                fire_gather(gn % 2, jn, rows0, semg0a, semg0b)

            return carry

        lax.fori_loop(0, _CPT // 2, pair, 0)
        wait2(scatter_halves((_NGR - 1) % 3, _G - 1, rows1,
                             sems1a, sems1b))

    @pl.when(cid == 0)
    def _():
        run(tg_hbm)

    @pl.when(cid == 1)
    def _():
        run(tu_hbm)

    plsc.subcore_barrier()

    def writeout(out_hbm):
        stripe_pieces(lambda row0, size: pltpu.sync_copy(
            acc_sh.at[pl.ds(row0, size)], out_hbm.at[pl.ds(row0, size)]))

    @pl.when(cid == 0)
    def _():
        writeout(accg_hbm)

    @pl.when(cid == 1)
    def _():
        writeout(accu_hbm)


def _prep_a_body(x_ref, w_ref, b_ref, h_ref, tu_ref):
    h = jnp.dot(x_ref[...], w_ref[...], preferred_element_type=jnp.float32)
    h = h + b_ref[...]
    nrm2 = jnp.sum(h * h, axis=1, keepdims=True)
    inv_n = lax.rsqrt(jnp.maximum(nrm2, 1e-24))
    h_ref[...] = h
    tu_ref[...] = h * inv_n


_prep_a_call = pl.pallas_call(
    _prep_a_body,
    out_shape=[
        jax.ShapeDtypeStruct((N, D), jnp.float32),
        jax.ShapeDtypeStruct((N, D), jnp.float32),
    ],
)


def _prep_b_body(h_ref, d0_ref, d1_ref, tg_ref, r_ref):
    deg = d0_ref[pl.ds(0, N)] + d1_ref[pl.ds(0, N)]
    r = lax.rsqrt(jnp.maximum(deg, 1.0))
    tg_ref[...] = h_ref[...] * r
    r_ref[...] = r


_prep_b_call = pl.pallas_call(
    _prep_b_body,
    out_shape=[
        jax.ShapeDtypeStruct((N, D), jnp.float32),
        jax.ShapeDtypeStruct((N, 1), jnp.float32),
    ],
)


def _finish_body(accg_ref, accu_ref, tu_ref, r_ref, x_ref, g_ref, be_ref,
                 out_ref, gcd_ref):
    agg = r_ref[...] * accg_ref[...]
    mean = jnp.mean(agg, axis=0, keepdims=True)
    cent = agg - mean
    var = jnp.mean(cent * cent, axis=0, keepdims=True)
    y = cent * lax.rsqrt(var + 1e-5) * g_ref[...] + be_ref[...]
    out_ref[...] = jnp.maximum(y, 0.0) + x_ref[...]
    cos_sum = jnp.sum(tu_ref[...] * accu_ref[...])
    gcd_ref[...] = jnp.reshape(1.0 - cos_sum * (1.0 / E), (1, 1))


_finish_call = pl.pallas_call(
    _finish_body,
    out_shape=[
        jax.ShapeDtypeStruct((N, D), jnp.float32),
        jax.ShapeDtypeStruct((1, 1), jnp.float32),
    ],
)


_NPAD = _EPAD - E
_PAD_SRC = jnp.asarray((np.arange(_NPAD, dtype=np.int32) * 131) % N)
_PAD_DST = jnp.asarray(N + (np.arange(_NPAD, dtype=np.int32) % _NJUNK))
_ZDEG = jnp.zeros((N + _NJUNK,), jnp.float32)
_ONES = jnp.ones((_CHUNK,), jnp.float32)
_ZROW = jnp.zeros((_STRIPE, D), jnp.float32)


def kernel(x, edge_index, W, b, gamma, beta):
    src = edge_index[0]
    dst = edge_index[1]
    src2 = jnp.concatenate([src, _PAD_SRC]).reshape(_NCHUNK, _CHUNK)
    dst2 = jnp.concatenate([dst, _PAD_DST]).reshape(_NCHUNK, _CHUNK)

    degp0, degp1 = _deg_kernel(dst2, _ZDEG, _ONES)
    h, tu = _prep_a_call(x, W, b.reshape(1, D))
    tg, r = _prep_b_call(h, degp0.reshape(N + _NJUNK, 1),
                         degp1.reshape(N + _NJUNK, 1))
    dst2h = dst2.reshape(_NCHUNK * 2, _CHUNK // 2)
    accg, accu = _scatter_kernel(tg, tu, src2, dst2h, _ZROW)
    out, gcd = _finish_call(accg, accu, tu, r, x,
                            gamma.reshape(1, D), beta.reshape(1, D))
    return out, gcd[0, 0]


# speedup vs baseline: 1.2037x; 1.0907x over previous
import functools

import jax
import jax.numpy as jnp
import numpy as np
from jax import lax
from jax.experimental import pallas as pl
from jax.experimental.pallas import tpu as pltpu
import jax.experimental.pallas.tpu_sc as plsc

N = 10000
E = 320000
D = 128

_NS = 16
_STRIPE = 632
_LAST = N - _STRIPE * (_NS - 1)
_CHUNK = 128
_CPT = 160
_NCHUNK = _CPT * _NS
_EPAD = _NCHUNK * _CHUNK
_NJUNK = 8
_CPD = _NCHUNK // 32
_G = 16
_NGR = _CPT // _G

_sc_mesh = plsc.VectorSubcoreMesh(core_axis_name="c", subcore_axis_name="s")


@functools.partial(
    pl.kernel,
    out_type=[
        jax.ShapeDtypeStruct((N + _NJUNK,), jnp.float32),
        jax.ShapeDtypeStruct((N + _NJUNK,), jnp.float32),
    ],
    mesh=_sc_mesh,
    scratch_types=[
        pltpu.VMEM_SHARED((N + _NJUNK,), jnp.float32),
        pltpu.VMEM((_CPD, _CHUNK), jnp.int32),
        pltpu.VMEM((_CHUNK,), jnp.float32),
        pltpu.SemaphoreType.DMA,
    ],
)
def _deg_kernel(dst2_hbm, zdeg_hbm, ones_hbm, degp0_hbm, degp1_hbm,
                deg_sh, dstv_all, onesv, semd):
    cid = lax.axis_index("c")
    sid = lax.axis_index("s")

    @pl.when(sid == 0)
    def _():
        pltpu.sync_copy(zdeg_hbm, deg_sh)

    pltpu.sync_copy(ones_hbm, onesv)
    rowbase = cid * (_NCHUNK // 2) + sid * _CPD
    pltpu.sync_copy(dst2_hbm.at[pl.ds(rowbase, _CPD)], dstv_all)
    plsc.subcore_barrier()

    def fire(i, carry):
        pltpu.async_copy(onesv, deg_sh.at[dstv_all.at[i]], semd, add=True)
        return carry

    lax.fori_loop(0, _CPD, fire, 0)

    def drain(i, carry):
        pltpu.make_async_copy(onesv, deg_sh.at[dstv_all.at[i]], semd).wait()
        return carry

    lax.fori_loop(0, _CPD, drain, 0)
    plsc.subcore_barrier()

    @pl.when(sid == 0)
    def _():
        @pl.when(cid == 0)
        def _():
            pltpu.sync_copy(deg_sh, degp0_hbm)

        @pl.when(cid == 1)
        def _():
            pltpu.sync_copy(deg_sh, degp1_hbm)


@functools.partial(
    pl.kernel,
    out_type=[
        jax.ShapeDtypeStruct((N, D), jnp.float32),
        jax.ShapeDtypeStruct((N, D), jnp.float32),
    ],
    mesh=_sc_mesh,
    scratch_types=[
        pltpu.VMEM_SHARED((N + _NJUNK, D), jnp.float32),
        pltpu.VMEM((2, _G, _CHUNK), jnp.int32),
        pltpu.VMEM((3, 2 * _G, _CHUNK // 2), jnp.int32),
        pltpu.VMEM((_CHUNK // 2, D), jnp.float32),
        pltpu.VMEM((_CHUNK // 2, D), jnp.float32),
        pltpu.VMEM((_CHUNK // 2, D), jnp.float32),
        pltpu.VMEM((_CHUNK // 2, D), jnp.float32),
        pltpu.SemaphoreType.DMA,
        pltpu.SemaphoreType.DMA,
        pltpu.SemaphoreType.DMA,
        pltpu.SemaphoreType.DMA,
        pltpu.SemaphoreType.DMA,
        pltpu.SemaphoreType.DMA,
        pltpu.SemaphoreType.DMA,
        pltpu.SemaphoreType.DMA,
        pltpu.SemaphoreType.DMA,
    ],
)
def _scatter_kernel(tg_hbm, tu_hbm, src2_hbm, dst2h_hbm, zrow_hbm,
                    accg_hbm, accu_hbm,
                    acc_sh, srcv3, dstv3, ra, rb, rc, rd,
                    sga, sgb, sgc, sgd,
                    ssa, ssb, ssc, ssd, semi):
    cid = lax.axis_index("c")
    sid = lax.axis_index("s")

    def stripe_pieces(fn):
        base = sid * _STRIPE
        for j in range(4):
            fn(base + j * 128, 128)

        @pl.when(sid < _NS - 1)
        def _():
            fn(base + 512, 120)

        @pl.when(sid == _NS - 1)
        def _():
            fn(base + 512, 8)

    stripe_pieces(lambda row0, size: pltpu.sync_copy(
        zrow_hbm.at[pl.ds(0, size)], acc_sh.at[pl.ds(row0, size)]))

    plsc.subcore_barrier()

    def run(table_hbm):
        tilebase = sid * _CPT
        H = _CHUNK // 2
        HU = 2 * _CPT
        bufs = (ra, rb, rc, rd)
        gsems = (sga, sgb, sgc, sgd)
        ssems = (ssa, ssb, ssc, ssd)

        def idx_load(grp, copy):
            gb = tilebase + grp * _G
            copy(src2_hbm.at[pl.ds(gb, _G)], srcv3.at[grp % 2])
            copy(dst2h_hbm.at[pl.ds(2 * gb, 2 * _G)], dstv3.at[grp % 3])

        def src_slice(v):
            grp = v // (2 * _G)
            return srcv3.at[grp % 2, lax.rem(v, 2 * _G) // 2,
                            pl.ds(lax.rem(v, 2) * H, H)]

        def dst_row(v):
            grp = v // (2 * _G)
            return dstv3.at[grp % 3, lax.rem(v, 2 * _G)]

        def wait_g(v, k):
            pltpu.make_async_copy(table_hbm.at[src_slice(v)], bufs[k],
                                  gsems[k]).wait()

        def fire_g(v, k):
            pltpu.async_copy(table_hbm.at[src_slice(v)], bufs[k], gsems[k])

        def fire_s(v, k):
            pltpu.async_copy(bufs[k], acc_sh.at[dst_row(v)], ssems[k],
                             add=True)

        def wait_s(v, k):
            pltpu.make_async_copy(bufs[k], acc_sh.at[dst_row(v)],
                                  ssems[k]).wait()

        idx_load(0, pltpu.sync_copy)
        idx_load(1, lambda s, d: pltpu.async_copy(s, d, semi))
        fire_g(0, 0)
        fire_g(1, 1)
        fire_g(2, 2)

        def unit(u, k):
            wait_g(u, k)
            fire_s(u, k)

            @pl.when(u + 3 < HU)
            def _():
                kn = (k + 3) % 4

                @pl.when(u > 0)
                def _():
                    wait_s(u - 1, kn)

                unext = u + 3
                gn = unext // (2 * _G)

                @pl.when(lax.rem(unext, 2 * _G) == 0)
                def _():
                    idx_load(gn,
                             lambda s, d: pltpu.make_async_copy(
                                 s, d, semi).wait())

                    @pl.when(gn + 1 < _NGR)
                    def _():
                        idx_load(gn + 1,
                                 lambda s, d: pltpu.async_copy(s, d, semi))

                fire_g(unext, kn)

        def quad(q, carry):
            for k in range(4):
                unit(4 * q + k, k)
            return carry

        lax.fori_loop(0, HU // 4, quad, 0)
        for v in range(HU - 4, HU):
            wait_s(v, v % 4)

    @pl.when(cid == 0)
    def _():
        run(tg_hbm)

    @pl.when(cid == 1)
    def _():
        run(tu_hbm)

    plsc.subcore_barrier()

    def writeout(out_hbm):
        stripe_pieces(lambda row0, size: pltpu.sync_copy(
            acc_sh.at[pl.ds(row0, size)], out_hbm.at[pl.ds(row0, size)]))

    @pl.when(cid == 0)
    def _():
        writeout(accg_hbm)

    @pl.when(cid == 1)
    def _():
        writeout(accu_hbm)


def _prep_a_body(x_ref, w_ref, b_ref, h_ref, tu_ref):
    h = jnp.dot(x_ref[...], w_ref[...], preferred_element_type=jnp.float32)
    h = h + b_ref[...]
    nrm2 = jnp.sum(h * h, axis=1, keepdims=True)
    inv_n = lax.rsqrt(jnp.maximum(nrm2, 1e-24))
    h_ref[...] = h
    tu_ref[...] = h * inv_n


_prep_a_call = pl.pallas_call(
    _prep_a_body,
    out_shape=[
        jax.ShapeDtypeStruct((N, D), jnp.float32),
        jax.ShapeDtypeStruct((N, D), jnp.float32),
    ],
)


def _prep_b_body(h_ref, d0_ref, d1_ref, tg_ref, r_ref):
    deg = d0_ref[pl.ds(0, N)] + d1_ref[pl.ds(0, N)]
    r = lax.rsqrt(jnp.maximum(deg, 1.0))
    tg_ref[...] = h_ref[...] * r
    r_ref[...] = r


_prep_b_call = pl.pallas_call(
    _prep_b_body,
    out_shape=[
        jax.ShapeDtypeStruct((N, D), jnp.float32),
        jax.ShapeDtypeStruct((N, 1), jnp.float32),
    ],
)


def _finish_body(accg_ref, accu_ref, tu_ref, r_ref, x_ref, g_ref, be_ref,
                 out_ref, gcd_ref):
    agg = r_ref[...] * accg_ref[...]
    mean = jnp.mean(agg, axis=0, keepdims=True)
    cent = agg - mean
    var = jnp.mean(cent * cent, axis=0, keepdims=True)
    y = cent * lax.rsqrt(var + 1e-5) * g_ref[...] + be_ref[...]
    out_ref[...] = jnp.maximum(y, 0.0) + x_ref[...]
    cos_sum = jnp.sum(tu_ref[...] * accu_ref[...])
    gcd_ref[...] = jnp.reshape(1.0 - cos_sum * (1.0 / E), (1, 1))


_finish_call = pl.pallas_call(
    _finish_body,
    out_shape=[
        jax.ShapeDtypeStruct((N, D), jnp.float32),
        jax.ShapeDtypeStruct((1, 1), jnp.float32),
    ],
)


_NPAD = _EPAD - E
_PAD_SRC = jnp.asarray((np.arange(_NPAD, dtype=np.int32) * 131) % N)
_PAD_DST = jnp.asarray(N + (np.arange(_NPAD, dtype=np.int32) % _NJUNK))
_ZDEG = jnp.zeros((N + _NJUNK,), jnp.float32)
_ONES = jnp.ones((_CHUNK,), jnp.float32)
_ZROW = jnp.zeros((_STRIPE, D), jnp.float32)


def kernel(x, edge_index, W, b, gamma, beta):
    src = edge_index[0]
    dst = edge_index[1]
    src2 = jnp.concatenate([src, _PAD_SRC]).reshape(_NCHUNK, _CHUNK)
    dst2 = jnp.concatenate([dst, _PAD_DST]).reshape(_NCHUNK, _CHUNK)

    degp0, degp1 = _deg_kernel(dst2, _ZDEG, _ONES)
    h, tu = _prep_a_call(x, W, b.reshape(1, D))
    tg, r = _prep_b_call(h, degp0.reshape(N + _NJUNK, 1),
                         degp1.reshape(N + _NJUNK, 1))
    dst2h = dst2.reshape(_NCHUNK * 2, _CHUNK // 2)
    accg, accu = _scatter_kernel(tg, tu, src2, dst2h, _ZROW)
    out, gcd = _finish_call(accg, accu, tu, r, x,
                            gamma.reshape(1, D), beta.reshape(1, D))
    return out, gcd[0, 0]


# speedup vs baseline: 1.2700x; 1.0550x over previous
import functools

import jax
import jax.numpy as jnp
import numpy as np
from jax import lax
from jax.experimental import pallas as pl
from jax.experimental.pallas import tpu as pltpu
import jax.experimental.pallas.tpu_sc as plsc

N = 10000
E = 320000
D = 128

_NS = 16
_STRIPE = 632
_LAST = N - _STRIPE * (_NS - 1)
_CHUNK = 128
_CPT = 160
_NCHUNK = _CPT * _NS
_EPAD = _NCHUNK * _CHUNK
_NJUNK = 8
_CPD = _NCHUNK // 32
_G = 8
_NGR = _CPT // _G

_sc_mesh = plsc.VectorSubcoreMesh(core_axis_name="c", subcore_axis_name="s")


@functools.partial(
    pl.kernel,
    out_type=[
        jax.ShapeDtypeStruct((N + _NJUNK,), jnp.float32),
        jax.ShapeDtypeStruct((N + _NJUNK,), jnp.float32),
    ],
    mesh=_sc_mesh,
    scratch_types=[
        pltpu.VMEM_SHARED((N + _NJUNK,), jnp.float32),
        pltpu.VMEM((_CPD, _CHUNK), jnp.int32),
        pltpu.VMEM((_CHUNK,), jnp.float32),
        pltpu.SemaphoreType.DMA,
    ],
)
def _deg_kernel(dst2_hbm, zdeg_hbm, ones_hbm, degp0_hbm, degp1_hbm,
                deg_sh, dstv_all, onesv, semd):
    cid = lax.axis_index("c")
    sid = lax.axis_index("s")

    @pl.when(sid == 0)
    def _():
        pltpu.sync_copy(zdeg_hbm, deg_sh)

    pltpu.sync_copy(ones_hbm, onesv)
    rowbase = cid * (_NCHUNK // 2) + sid * _CPD
    pltpu.sync_copy(dst2_hbm.at[pl.ds(rowbase, _CPD)], dstv_all)
    plsc.subcore_barrier()

    def fire(i, carry):
        pltpu.async_copy(onesv, deg_sh.at[dstv_all.at[i]], semd, add=True)
        return carry

    lax.fori_loop(0, _CPD, fire, 0)

    def drain(i, carry):
        pltpu.make_async_copy(onesv, deg_sh.at[dstv_all.at[i]], semd).wait()
        return carry

    lax.fori_loop(0, _CPD, drain, 0)
    plsc.subcore_barrier()

    @pl.when(sid == 0)
    def _():
        @pl.when(cid == 0)
        def _():
            pltpu.sync_copy(deg_sh, degp0_hbm)

        @pl.when(cid == 1)
        def _():
            pltpu.sync_copy(deg_sh, degp1_hbm)


@functools.partial(
    pl.kernel,
    out_type=[
        jax.ShapeDtypeStruct((N, D), jnp.float32),
        jax.ShapeDtypeStruct((N, D), jnp.float32),
    ],
    mesh=_sc_mesh,
    scratch_types=[
        pltpu.VMEM_SHARED((N + _NJUNK, D), jnp.float32),
        pltpu.VMEM((3, _G, _CHUNK), jnp.int32),
        pltpu.VMEM((3, 2 * _G, _CHUNK // 2), jnp.int32),
        pltpu.VMEM((_CHUNK // 2, D), jnp.float32),
        pltpu.VMEM((_CHUNK // 2, D), jnp.float32),
        pltpu.VMEM((_CHUNK // 2, D), jnp.float32),
        pltpu.VMEM((_CHUNK // 2, D), jnp.float32),
        pltpu.VMEM((_CHUNK // 2, D), jnp.float32),
        pltpu.SemaphoreType.DMA,
        pltpu.SemaphoreType.DMA,
        pltpu.SemaphoreType.DMA,
        pltpu.SemaphoreType.DMA,
        pltpu.SemaphoreType.DMA,
        pltpu.SemaphoreType.DMA,
        pltpu.SemaphoreType.DMA,
        pltpu.SemaphoreType.DMA,
        pltpu.SemaphoreType.DMA,
        pltpu.SemaphoreType.DMA,
        pltpu.SemaphoreType.DMA,
    ],
)
def _scatter_kernel(tg_hbm, tu_hbm, src2_hbm, dst2h_hbm, zrow_hbm,
                    accg_hbm, accu_hbm,
                    acc_sh, srcv3, dstv3, ra, rb, rc, rd, re,
                    sga, sgb, sgc, sgd, sge,
                    ssa, ssb, ssc, ssd, sse, semi):
    cid = lax.axis_index("c")
    sid = lax.axis_index("s")

    def stripe_pieces(fn):
        base = sid * _STRIPE
        for j in range(4):
            fn(base + j * 128, 128)

        @pl.when(sid < _NS - 1)
        def _():
            fn(base + 512, 120)

        @pl.when(sid == _NS - 1)
        def _():
            fn(base + 512, 8)

    stripe_pieces(lambda row0, size: pltpu.sync_copy(
        zrow_hbm.at[pl.ds(0, size)], acc_sh.at[pl.ds(row0, size)]))

    plsc.subcore_barrier()

    def run(table_hbm):
        tilebase = sid * _CPT
        H = _CHUNK // 2
        HU = 2 * _CPT
        bufs = (ra, rb, rc, rd, re)
        gsems = (sga, sgb, sgc, sgd, sge)
        ssems = (ssa, ssb, ssc, ssd, sse)

        def idx_load(grp, copy):
            gb = tilebase + grp * _G
            copy(src2_hbm.at[pl.ds(gb, _G)], srcv3.at[grp % 3])
            copy(dst2h_hbm.at[pl.ds(2 * gb, 2 * _G)], dstv3.at[grp % 3])

        def src_slice(v):
            grp = v // (2 * _G)
            return srcv3.at[grp % 3, lax.rem(v, 2 * _G) // 2,
                            pl.ds(lax.rem(v, 2) * H, H)]

        def dst_row(v):
            grp = v // (2 * _G)
            return dstv3.at[grp % 3, lax.rem(v, 2 * _G)]

        def wait_g(v, k):
            pltpu.make_async_copy(table_hbm.at[src_slice(v)], bufs[k],
                                  gsems[k]).wait()

        def fire_g(v, k):
            pltpu.async_copy(table_hbm.at[src_slice(v)], bufs[k], gsems[k])

        def fire_s(v, k):
            pltpu.async_copy(bufs[k], acc_sh.at[dst_row(v)], ssems[k],
                             add=True)

        def wait_s(v, k):
            pltpu.make_async_copy(bufs[k], acc_sh.at[dst_row(v)],
                                  ssems[k]).wait()

        idx_load(0, pltpu.sync_copy)
        idx_load(1, lambda s, d: pltpu.async_copy(s, d, semi))
        fire_g(0, 0)
        fire_g(1, 1)
        fire_g(2, 2)
        fire_g(3, 3)

        def unit(u, k):
            wait_g(u, k)
            fire_s(u, k)

            @pl.when(u + 4 < HU)
            def _():
                kn = (k + 4) % 5

                @pl.when(u > 0)
                def _():
                    wait_s(u - 1, kn)

                unext = u + 4
                gn = unext // (2 * _G)

                @pl.when(lax.rem(unext, 2 * _G) == 0)
                def _():
                    idx_load(gn,
                             lambda s, d: pltpu.make_async_copy(
                                 s, d, semi).wait())

                    @pl.when(gn + 1 < _NGR)
                    def _():
                        idx_load(gn + 1,
                                 lambda s, d: pltpu.async_copy(s, d, semi))

                fire_g(unext, kn)

        def quint(q, carry):
            for k in range(5):
                unit(5 * q + k, k)
            return carry

        lax.fori_loop(0, HU // 5, quint, 0)
        for v in range(HU - 5, HU):
            wait_s(v, v % 5)

    @pl.when(cid == 0)
    def _():
        run(tg_hbm)

    @pl.when(cid == 1)
    def _():
        run(tu_hbm)

    plsc.subcore_barrier()

    def writeout(out_hbm):
        stripe_pieces(lambda row0, size: pltpu.sync_copy(
            acc_sh.at[pl.ds(row0, size)], out_hbm.at[pl.ds(row0, size)]))

    @pl.when(cid == 0)
    def _():
        writeout(accg_hbm)

    @pl.when(cid == 1)
    def _():
        writeout(accu_hbm)


def _prep_a_body(x_ref, w_ref, b_ref, h_ref, tu_ref):
    h = jnp.dot(x_ref[...], w_ref[...], preferred_element_type=jnp.float32)
    h = h + b_ref[...]
    nrm2 = jnp.sum(h * h, axis=1, keepdims=True)
    inv_n = lax.rsqrt(jnp.maximum(nrm2, 1e-24))
    h_ref[...] = h
    tu_ref[...] = h * inv_n


_prep_a_call = pl.pallas_call(
    _prep_a_body,
    out_shape=[
        jax.ShapeDtypeStruct((N, D), jnp.float32),
        jax.ShapeDtypeStruct((N, D), jnp.float32),
    ],
)


def _prep_b_body(h_ref, d0_ref, d1_ref, tg_ref, r_ref):
    deg = d0_ref[pl.ds(0, N)] + d1_ref[pl.ds(0, N)]
    r = lax.rsqrt(jnp.maximum(deg, 1.0))
    tg_ref[...] = h_ref[...] * r
    r_ref[...] = r


_prep_b_call = pl.pallas_call(
    _prep_b_body,
    out_shape=[
        jax.ShapeDtypeStruct((N, D), jnp.float32),
        jax.ShapeDtypeStruct((N, 1), jnp.float32),
    ],
)


def _finish_body(accg_ref, accu_ref, tu_ref, r_ref, x_ref, g_ref, be_ref,
                 out_ref, gcd_ref):
    agg = r_ref[...] * accg_ref[...]
    mean = jnp.mean(agg, axis=0, keepdims=True)
    cent = agg - mean
    var = jnp.mean(cent * cent, axis=0, keepdims=True)
    y = cent * lax.rsqrt(var + 1e-5) * g_ref[...] + be_ref[...]
    out_ref[...] = jnp.maximum(y, 0.0) + x_ref[...]
    cos_sum = jnp.sum(tu_ref[...] * accu_ref[...])
    gcd_ref[...] = jnp.reshape(1.0 - cos_sum * (1.0 / E), (1, 1))


_finish_call = pl.pallas_call(
    _finish_body,
    out_shape=[
        jax.ShapeDtypeStruct((N, D), jnp.float32),
        jax.ShapeDtypeStruct((1, 1), jnp.float32),
    ],
)


_NPAD = _EPAD - E
_PAD_SRC = jnp.asarray((np.arange(_NPAD, dtype=np.int32) * 131) % N)
_PAD_DST = jnp.asarray(N + (np.arange(_NPAD, dtype=np.int32) % _NJUNK))
_ZDEG = jnp.zeros((N + _NJUNK,), jnp.float32)
_ONES = jnp.ones((_CHUNK,), jnp.float32)
_ZROW = jnp.zeros((_STRIPE, D), jnp.float32)


def kernel(x, edge_index, W, b, gamma, beta):
    src = edge_index[0]
    dst = edge_index[1]
    src2 = jnp.concatenate([src, _PAD_SRC]).reshape(_NCHUNK, _CHUNK)
    dst2 = jnp.concatenate([dst, _PAD_DST]).reshape(_NCHUNK, _CHUNK)

    degp0, degp1 = _deg_kernel(dst2, _ZDEG, _ONES)
    h, tu = _prep_a_call(x, W, b.reshape(1, D))
    tg, r = _prep_b_call(h, degp0.reshape(N + _NJUNK, 1),
                         degp1.reshape(N + _NJUNK, 1))
    dst2h = dst2.reshape(_NCHUNK * 2, _CHUNK // 2)
    accg, accu = _scatter_kernel(tg, tu, src2, dst2h, _ZROW)
    out, gcd = _finish_call(accg, accu, tu, r, x,
                            gamma.reshape(1, D), beta.reshape(1, D))
    return out, gcd[0, 0]
